# baseline (speedup 1.0000x reference)
import functools

import jax
import jax.numpy as jnp
import numpy as np
from jax.experimental import pallas as pl

E = 820
H = 41
D = 20
CB = 7
SB = 2
WIN = 5
TOPK = 16
INTER = 2304
EPS = 1e-6
B = 2
L = 2044
LC = L // CB
NBLK = L // SB
NSEL = TOPK * SB
NKEY = LC * SB
ROWS = B * L
RT = 584
SCALE = 1.0 / float(np.sqrt(D))

_DOT = functools.partial(jax.lax.dot_general,
                         preferred_element_type=jnp.float32)
CN = (((1,), (0,)), ((), ()))
CT = (((1,), (1,)), ((), ()))
CA = (((0,), (0,)), ((), ()))


def _bf(t):
    return t.astype(jnp.bfloat16)


def _proj_kernel(x_ref, nw_ref, w_ref, b_ref, o_ref):
    x = x_ref[0]
    ms = jnp.mean(x * x, axis=1, keepdims=True)
    h = x * jax.lax.rsqrt(ms + EPS) * nw_ref[...]
    if w_ref.dtype == jnp.bfloat16:
        h = _bf(h)
    o_ref[0] = (_DOT(w_ref[...], h, CT) + b_ref[...]).astype(o_ref.dtype)


def _proj(x, nw, w, b, out_dtype):
    return pl.pallas_call(
        _proj_kernel,
        grid=(B,),
        in_specs=[pl.BlockSpec((1, L, E), lambda i: (i, 0, 0)),
                  pl.BlockSpec((1, E), lambda i: (0, 0)),
                  pl.BlockSpec((E, E), lambda i: (0, 0)),
                  pl.BlockSpec((E, 1), lambda i: (0, 0))],
        out_specs=[pl.BlockSpec((1, E, L), lambda i: (i, 0, 0))],
        out_shape=[jax.ShapeDtypeStruct((B, E, L), out_dtype)],
    )(x, nw.reshape(1, E), w, b.reshape(E, 1))[0]


def _attn_kernel(qT_ref, kT_ref, vT_ref, kbT_ref, vbT_ref,
                 wc1_ref, bc1_ref, wc2_ref, bc2_ref, wg_ref, bg_ref,
                 out_ref):
    qT = qT_ref[0, 0]
    qTb = _bf(qT)
    kT = kT_ref[0, 0]
    vT = vT_ref[0, 0]

    h1k = jnp.maximum(_DOT(wc1_ref[...], kbT_ref[0, 0], CN) + bc1_ref[...], 0.0)
    kcT = _DOT(wc2_ref[...], h1k, CN) + bc2_ref[...]
    h1v = jnp.maximum(_DOT(wc1_ref[...], vbT_ref[0, 0], CN) + bc1_ref[...], 0.0)
    vcT = _DOT(wc2_ref[...], h1v, CN) + bc2_ref[...]

    sT = _DOT(kcT, qT, CA) * SCALE
    m = jnp.max(sT, axis=0, keepdims=True)
    e = jnp.exp(sT - m)
    aT = e * jax.lax.reciprocal(jnp.sum(e, axis=0, keepdims=True))
    attn_compT = _DOT(_bf(vcT), _bf(aT), CN)
    bs_col = jnp.sum(aT, axis=1, keepdims=True)

    jj = jax.lax.broadcasted_iota(jnp.int32, (LC, LC), 0)
    ii = jax.lax.broadcasted_iota(jnp.int32, (LC, LC), 1)
    bs_row = jnp.transpose(bs_col)
    cmp = (bs_row > bs_col) | ((bs_row == bs_col) & (ii < jj))
    rank_col = jnp.sum(cmp.astype(jnp.int32), axis=1, keepdims=True)

    ll = jax.lax.broadcasted_iota(jnp.int32, (NKEY, LC), 0)
    jj2 = jax.lax.broadcasted_iota(jnp.int32, (NKEY, LC), 1)
    expand = ((ll // SB) == jj2).astype(jnp.float32)
    par = jax.lax.broadcasted_iota(jnp.int32, (NKEY, 1), 0) % SB
    rkey = (SB * _DOT(expand, rank_col.astype(jnp.float32), CN)
            + par.astype(jnp.float32))
    mm = jax.lax.broadcasted_iota(jnp.int32, (1, NSEL), 1).astype(jnp.float32)
    g2 = (rkey == mm).astype(jnp.float32)

    kselT = _DOT(kT[:, :NKEY], g2, CN)
    vselT = _DOT(vT[:, :NKEY], _bf(g2), CN)

    s2T = _DOT(_bf(kselT), qTb, CA) * SCALE
    m2 = jnp.max(s2T, axis=0, keepdims=True)
    e2 = jnp.exp(s2T - m2)
    r2 = jax.lax.reciprocal(jnp.sum(e2, axis=0, keepdims=True))
    attn_selT = _DOT(_bf(vselT), _bf(e2), CN) * r2

    kwT = _bf(kT[:, L - WIN:])
    vwT = vT[:, L - WIN:]
    s3T = _DOT(kwT, qTb, CA) * SCALE
    m3 = jnp.max(s3T, axis=0, keepdims=True)
    e3 = jnp.exp(s3T - m3)
    r3 = jax.lax.reciprocal(jnp.sum(e3, axis=0, keepdims=True))
    attn_winT = _DOT(vwT, _bf(e3), CN) * r3

    glT = _DOT(_bf(wg_ref[...]), qTb, CN) + bg_ref[...]
    mg = jnp.max(glT, axis=0, keepdims=True)
    eg = jnp.exp(glT - mg)
    gw = eg * jax.lax.reciprocal(jnp.sum(eg, axis=0, keepdims=True))
    out_ref[0, 0] = (gw[0:1, :] * attn_compT + gw[1:2, :] * attn_selT
                     + gw[2:3, :] * attn_winT)


def _attention(qT, kT, vT, kbT, vbT, Wc1, bc1, Wc2, bc2, Wg, bg):
    head_spec = pl.BlockSpec((1, 1, D, L), lambda b, h: (b, h, 0, 0))
    blk_spec = pl.BlockSpec((1, 1, CB * D, LC), lambda b, h: (b, h, 0, 0))

    def full(shape):
        return pl.BlockSpec(shape, lambda b, h: (0,) * len(shape))

    return pl.pallas_call(
        _attn_kernel,
        grid=(B, H),
        in_specs=[head_spec, head_spec, head_spec, blk_spec, blk_spec,
                  full((D // 2, CB * D)), full((D // 2, 1)),
                  full((D, D // 2)), full((D, 1)),
                  full((3, D)), full((3, 1))],
        out_specs=[head_spec],
        out_shape=[jax.ShapeDtypeStruct((B, H, D, L), jnp.float32)],
    )(qT.reshape(B, H, D, L), kT.reshape(B, H, D, L), vT.reshape(B, H, D, L),
      kbT, vbT, Wc1, bc1.reshape(D // 2, 1), Wc2, bc2.reshape(D, 1),
      Wg, bg.reshape(3, 1))[0]


def _mlp_kernel(x_ref, a_ref, nw_ref, wg_ref, wu_ref, wd_ref, o_ref):
    x2 = x_ref[...] + a_ref[...]
    ms = jnp.mean(x2 * x2, axis=1, keepdims=True)
    h = _bf(x2 * jax.lax.rsqrt(ms + EPS) * nw_ref[...])
    g = jnp.dot(h, wg_ref[...], preferred_element_type=jnp.float32)
    u = jnp.dot(h, wu_ref[...], preferred_element_type=jnp.float32)
    act = _bf(g * jax.nn.sigmoid(g) * u)
    o_ref[...] = jnp.dot(act, wd_ref[...], preferred_element_type=jnp.float32) + x2


def _mlp(xf, af, nw, wgT, wuT, wdT):
    grid = (ROWS // RT,)
    row_spec = pl.BlockSpec((RT, E), lambda i: (i, 0))
    return pl.pallas_call(
        _mlp_kernel,
        grid=grid,
        in_specs=[row_spec, row_spec,
                  pl.BlockSpec((1, E), lambda i: (0, 0)),
                  pl.BlockSpec((E, INTER), lambda i: (0, 0)),
                  pl.BlockSpec((E, INTER), lambda i: (0, 0)),
                  pl.BlockSpec((INTER, E), lambda i: (0, 0))],
        out_specs=[row_spec],
        out_shape=[jax.ShapeDtypeStruct((ROWS, E), jnp.float32)],
    )(xf, af, nw.reshape(1, E), wgT, wuT, wdT)[0]


def kernel(x, attn_norm_w, Wq, bq, Wk, bk, Wv, bv, Wc1, bc1, Wc2, bc2, Wg, bg,
           mlp_norm_w, W_gate, W_up, W_down):
    qT = _proj(x, attn_norm_w, Wq, bq, jnp.float32)
    kT = _proj(x, attn_norm_w, Wk, bk, jnp.float32)
    vT = _proj(x, attn_norm_w, _bf(Wv), bv, jnp.bfloat16)

    kbT = (kT.reshape(B, H, D, LC, CB).transpose(0, 1, 4, 2, 3)
             .reshape(B, H, CB * D, LC))
    vbT = (vT.reshape(B, H, D, LC, CB).transpose(0, 1, 4, 2, 3)
             .reshape(B, H, CB * D, LC))

    attnT = _attention(qT, kT, vT, kbT, vbT, Wc1, bc1, Wc2, bc2, Wg, bg)
    af = attnT.reshape(B, E, L).transpose(0, 2, 1).reshape(ROWS, E)

    xf = x.reshape(ROWS, E)
    out = _mlp(xf, af, mlp_norm_w, _bf(W_gate.T), _bf(W_up.T), _bf(W_down.T))
    return out.reshape(B, L, E)


# speedup vs baseline: 5.8091x; 5.8091x over previous
import functools

import jax
import jax.numpy as jnp
import numpy as np
from jax.experimental import pallas as pl

E = 820
H = 41
D = 20
CB = 7
SB = 2
WIN = 5
TOPK = 16
INTER = 2304
EPS = 1e-6
B = 2
L = 2044
LC = L // CB
NBLK = L // SB
NSEL = TOPK * SB
NKEY = LC * SB
ROWS = B * L
RT = 584
SCALE = 1.0 / float(np.sqrt(D))

_DOT = functools.partial(jax.lax.dot_general,
                         preferred_element_type=jnp.float32)
CN = (((1,), (0,)), ((), ()))
CT = (((1,), (1,)), ((), ()))
CA = (((0,), (0,)), ((), ()))


def _bf(t):
    return t.astype(jnp.bfloat16)


def _proj_kernel(x_ref, nw_ref, w_ref, b_ref, o_ref):
    x = x_ref[0]
    ms = jnp.mean(x * x, axis=1, keepdims=True)
    h = x * jax.lax.rsqrt(ms + EPS) * nw_ref[...]
    if w_ref.dtype == jnp.bfloat16:
        h = _bf(h)
    o_ref[0] = (_DOT(w_ref[...], h, CT) + b_ref[...]).astype(o_ref.dtype)


def _proj(x, nw, w, b, out_dtype):
    return pl.pallas_call(
        _proj_kernel,
        grid=(B,),
        in_specs=[pl.BlockSpec((1, L, E), lambda i: (i, 0, 0)),
                  pl.BlockSpec((1, E), lambda i: (0, 0)),
                  pl.BlockSpec((E, E), lambda i: (0, 0)),
                  pl.BlockSpec((E, 1), lambda i: (0, 0))],
        out_specs=[pl.BlockSpec((1, E, L), lambda i: (i, 0, 0))],
        out_shape=[jax.ShapeDtypeStruct((B, E, L), out_dtype)],
    )(x, nw.reshape(1, E), w, b.reshape(E, 1))[0]


def _attn_kernel(qT_ref, kT_ref, vT_ref, kbT_ref, vbT_ref,
                 wc1_ref, bc1_ref, wc2_ref, bc2_ref, wg_ref, bg_ref,
                 out_ref):
    qT = qT_ref[0, 0]
    qTb = _bf(qT)
    kT = kT_ref[0, 0]
    vT = vT_ref[0, 0]

    h1k = jnp.maximum(_DOT(kbT_ref[0, 0], wc1_ref[...], CT) + bc1_ref[...], 0.0)
    kc = _DOT(h1k, wc2_ref[...], CT) + bc2_ref[...]
    h1v = jnp.maximum(_DOT(vbT_ref[0, 0], wc1_ref[...], CT) + bc1_ref[...], 0.0)
    vc = _DOT(h1v, wc2_ref[...], CT) + bc2_ref[...]

    sT = _DOT(kc, qT, CN) * SCALE
    m = jnp.max(sT, axis=0, keepdims=True)
    e = jnp.exp(sT - m)
    aT = e * jax.lax.reciprocal(jnp.sum(e, axis=0, keepdims=True))
    attn_compT = _DOT(_bf(vc), _bf(aT), CA)
    bs_col = jnp.sum(aT, axis=1, keepdims=True)

    jj = jax.lax.broadcasted_iota(jnp.int32, (LC, LC), 0)
    ii = jax.lax.broadcasted_iota(jnp.int32, (LC, LC), 1)
    bs_row = jnp.transpose(bs_col)
    cmp = (bs_row > bs_col) | ((bs_row == bs_col) & (ii < jj))
    rank_col = jnp.sum(cmp.astype(jnp.int32), axis=1, keepdims=True)

    ll = jax.lax.broadcasted_iota(jnp.int32, (NKEY, LC), 0)
    jj2 = jax.lax.broadcasted_iota(jnp.int32, (NKEY, LC), 1)
    expand = ((ll // SB) == jj2).astype(jnp.float32)
    par = jax.lax.broadcasted_iota(jnp.int32, (NKEY, 1), 0) % SB
    rkey = (SB * _DOT(expand, rank_col.astype(jnp.float32), CN)
            + par.astype(jnp.float32))
    mm = jax.lax.broadcasted_iota(jnp.int32, (1, NSEL), 1).astype(jnp.float32)
    g2 = (rkey == mm).astype(jnp.float32)

    kselT = _DOT(kT[:, :NKEY], g2, CN)
    vselT = _DOT(vT[:, :NKEY], _bf(g2), CN)

    s2T = _DOT(_bf(kselT), qTb, CA) * SCALE
    m2 = jnp.max(s2T, axis=0, keepdims=True)
    e2 = jnp.exp(s2T - m2)
    r2 = jax.lax.reciprocal(jnp.sum(e2, axis=0, keepdims=True))
    attn_selT = _DOT(_bf(vselT), _bf(e2), CN) * r2

    kwT = _bf(kT[:, L - WIN:])
    vwT = vT[:, L - WIN:]
    s3T = _DOT(kwT, qTb, CA) * SCALE
    m3 = jnp.max(s3T, axis=0, keepdims=True)
    e3 = jnp.exp(s3T - m3)
    r3 = jax.lax.reciprocal(jnp.sum(e3, axis=0, keepdims=True))
    attn_winT = _DOT(vwT, _bf(e3), CN) * r3

    glT = _DOT(_bf(wg_ref[...]), qTb, CN) + bg_ref[...]
    mg = jnp.max(glT, axis=0, keepdims=True)
    eg = jnp.exp(glT - mg)
    gw = eg * jax.lax.reciprocal(jnp.sum(eg, axis=0, keepdims=True))
    out_ref[0, 0] = (gw[0:1, :] * attn_compT + gw[1:2, :] * attn_selT
                     + gw[2:3, :] * attn_winT)


def _attention(qT, kT, vT, kbT, vbT, Wc1, bc1, Wc2, bc2, Wg, bg):
    head_spec = pl.BlockSpec((1, 1, D, L), lambda b, h: (b, h, 0, 0))
    blk_spec = pl.BlockSpec((1, 1, LC, CB * D), lambda b, h: (b, h, 0, 0))

    def full(shape):
        return pl.BlockSpec(shape, lambda b, h: (0,) * len(shape))

    return pl.pallas_call(
        _attn_kernel,
        grid=(B, H),
        in_specs=[head_spec, head_spec, head_spec, blk_spec, blk_spec,
                  full((D // 2, CB * D)), full((1, D // 2)),
                  full((D, D // 2)), full((1, D)),
                  full((3, D)), full((3, 1))],
        out_specs=[head_spec],
        out_shape=[jax.ShapeDtypeStruct((B, H, D, L), jnp.float32)],
    )(qT.reshape(B, H, D, L), kT.reshape(B, H, D, L), vT.reshape(B, H, D, L),
      kbT, vbT, Wc1, bc1.reshape(1, D // 2), Wc2, bc2.reshape(1, D),
      Wg, bg.reshape(3, 1))[0]


def _mlp_kernel(x_ref, a_ref, nw_ref, wg_ref, wu_ref, wd_ref, o_ref):
    x2 = x_ref[...] + a_ref[...]
    ms = jnp.mean(x2 * x2, axis=1, keepdims=True)
    h = _bf(x2 * jax.lax.rsqrt(ms + EPS) * nw_ref[...])
    g = jnp.dot(h, wg_ref[...], preferred_element_type=jnp.float32)
    u = jnp.dot(h, wu_ref[...], preferred_element_type=jnp.float32)
    act = _bf(g * jax.nn.sigmoid(g) * u)
    o_ref[...] = jnp.dot(act, wd_ref[...], preferred_element_type=jnp.float32) + x2


def _mlp(xf, af, nw, wgT, wuT, wdT):
    grid = (ROWS // RT,)
    row_spec = pl.BlockSpec((RT, E), lambda i: (i, 0))
    return pl.pallas_call(
        _mlp_kernel,
        grid=grid,
        in_specs=[row_spec, row_spec,
                  pl.BlockSpec((1, E), lambda i: (0, 0)),
                  pl.BlockSpec((E, INTER), lambda i: (0, 0)),
                  pl.BlockSpec((E, INTER), lambda i: (0, 0)),
                  pl.BlockSpec((INTER, E), lambda i: (0, 0))],
        out_specs=[row_spec],
        out_shape=[jax.ShapeDtypeStruct((ROWS, E), jnp.float32)],
    )(xf, af, nw.reshape(1, E), wgT, wuT, wdT)[0]


def kernel(x, attn_norm_w, Wq, bq, Wk, bk, Wv, bv, Wc1, bc1, Wc2, bc2, Wg, bg,
           mlp_norm_w, W_gate, W_up, W_down):
    qT = _proj(x, attn_norm_w, Wq, bq, jnp.float32)
    kT = _proj(x, attn_norm_w, Wk, bk, jnp.float32)
    vT = _proj(x, attn_norm_w, _bf(Wv), bv, jnp.bfloat16)

    kbT = (kT.reshape(B, H, D, L).transpose(0, 1, 3, 2)
             .reshape(B, H, LC, CB * D))
    vbT = (vT.reshape(B, H, D, L).transpose(0, 1, 3, 2)
             .reshape(B, H, LC, CB * D))

    attnT = _attention(qT, kT, vT, kbT, vbT, Wc1, bc1, Wc2, bc2, Wg, bg)
    af = attnT.reshape(B, E, L).transpose(0, 2, 1).reshape(ROWS, E)

    xf = x.reshape(ROWS, E)
    out = _mlp(xf, af, mlp_norm_w, _bf(W_gate.T), _bf(W_up.T), _bf(W_down.T))
    return out.reshape(B, L, E)


# speedup vs baseline: 5.9100x; 1.0174x over previous
import functools

import jax
import jax.numpy as jnp
import numpy as np
from jax.experimental import pallas as pl

E = 820
H = 41
D = 20
CB = 7
SB = 2
WIN = 5
TOPK = 16
INTER = 2304
EPS = 1e-6
B = 2
L = 2044
LC = L // CB
NBLK = L // SB
NSEL = TOPK * SB
NKEY = LC * SB
ROWS = B * L
RT = 584
SCALE = 1.0 / float(np.sqrt(D))

_DOT = functools.partial(jax.lax.dot_general,
                         preferred_element_type=jnp.float32)
CN = (((1,), (0,)), ((), ()))
CT = (((1,), (1,)), ((), ()))
CA = (((0,), (0,)), ((), ()))


def _bf(t):
    return t.astype(jnp.bfloat16)


def _proj_kernel(x_ref, nw_ref, w_ref, b_ref, o_ref):
    x = x_ref[0]
    ms = jnp.mean(x * x, axis=1, keepdims=True)
    h = x * jax.lax.rsqrt(ms + EPS) * nw_ref[...]
    if w_ref.dtype == jnp.bfloat16:
        h = _bf(h)
    o_ref[0] = (_DOT(w_ref[...], h, CT) + b_ref[...]).astype(o_ref.dtype)


def _proj(x, nw, w, b, out_dtype):
    return pl.pallas_call(
        _proj_kernel,
        grid=(B,),
        in_specs=[pl.BlockSpec((1, L, E), lambda i: (i, 0, 0)),
                  pl.BlockSpec((1, E), lambda i: (0, 0)),
                  pl.BlockSpec((E, E), lambda i: (0, 0)),
                  pl.BlockSpec((E, 1), lambda i: (0, 0))],
        out_specs=[pl.BlockSpec((1, E, L), lambda i: (i, 0, 0))],
        out_shape=[jax.ShapeDtypeStruct((B, E, L), out_dtype)],
    )(x, nw.reshape(1, E), w, b.reshape(E, 1))[0]


def _attn_kernel(qT_ref, kT_ref, vT_ref, kbT_ref, vbT_ref,
                 wc1_ref, bc1_ref, wc2_ref, bc2_ref, wg_ref, bg_ref,
                 out_ref):
  for _b in range(B):
      qT = qT_ref[_b, 0]
      qTb = _bf(qT)
      kT = kT_ref[_b, 0]
      vT = vT_ref[_b, 0]

      h1k = jnp.maximum(_DOT(kbT_ref[_b, 0], wc1_ref[...], CT) + bc1_ref[...], 0.0)
      kc = _DOT(h1k, wc2_ref[...], CT) + bc2_ref[...]
      h1v = jnp.maximum(_DOT(vbT_ref[_b, 0], wc1_ref[...], CT) + bc1_ref[...], 0.0)
      vc = _DOT(h1v, wc2_ref[...], CT) + bc2_ref[...]

      sT = _DOT(kc, qT, CN) * SCALE
      m = jnp.max(sT, axis=0, keepdims=True)
      e = jnp.exp(sT - m)
      aT = e * jax.lax.reciprocal(jnp.sum(e, axis=0, keepdims=True))
      attn_compT = _DOT(_bf(vc), _bf(aT), CA)
      bs_col = jnp.sum(aT, axis=1, keepdims=True)

      jj = jax.lax.broadcasted_iota(jnp.int32, (LC, LC), 0)
      ii = jax.lax.broadcasted_iota(jnp.int32, (LC, LC), 1)
      bs_row = jnp.transpose(bs_col)
      cmp = (bs_row > bs_col) | ((bs_row == bs_col) & (ii < jj))
      rank_col = jnp.sum(cmp.astype(jnp.int32), axis=1, keepdims=True)

      ll = jax.lax.broadcasted_iota(jnp.int32, (NKEY, LC), 0)
      jj2 = jax.lax.broadcasted_iota(jnp.int32, (NKEY, LC), 1)
      expand = ((ll // SB) == jj2).astype(jnp.float32)
      par = jax.lax.broadcasted_iota(jnp.int32, (NKEY, 1), 0) % SB
      rkey = (SB * _DOT(expand, rank_col.astype(jnp.float32), CN)
              + par.astype(jnp.float32))
      mm = jax.lax.broadcasted_iota(jnp.int32, (1, NSEL), 1).astype(jnp.float32)
      g2 = (rkey == mm).astype(jnp.float32)

      kselT = _DOT(kT[:, :NKEY], g2, CN)
      vselT = _DOT(vT[:, :NKEY], _bf(g2), CN)

      s2T = _DOT(_bf(kselT), qTb, CA) * SCALE
      m2 = jnp.max(s2T, axis=0, keepdims=True)
      e2 = jnp.exp(s2T - m2)
      r2 = jax.lax.reciprocal(jnp.sum(e2, axis=0, keepdims=True))
      attn_selT = _DOT(_bf(vselT), _bf(e2), CN) * r2

      kwT = _bf(kT[:, L - WIN:])
      vwT = vT[:, L - WIN:]
      s3T = _DOT(kwT, qTb, CA) * SCALE
      m3 = jnp.max(s3T, axis=0, keepdims=True)
      e3 = jnp.exp(s3T - m3)
      r3 = jax.lax.reciprocal(jnp.sum(e3, axis=0, keepdims=True))
      attn_winT = _DOT(vwT, _bf(e3), CN) * r3

      glT = _DOT(_bf(wg_ref[...]), qTb, CN) + bg_ref[...]
      mg = jnp.max(glT, axis=0, keepdims=True)
      eg = jnp.exp(glT - mg)
      gw = eg * jax.lax.reciprocal(jnp.sum(eg, axis=0, keepdims=True))
      out_ref[_b, 0] = (gw[0:1, :] * attn_compT + gw[1:2, :] * attn_selT
                       + gw[2:3, :] * attn_winT)


def _attention(qT, kT, vT, kbT, vbT, Wc1, bc1, Wc2, bc2, Wg, bg):
    head_spec = pl.BlockSpec((B, 1, D, L), lambda h: (0, h, 0, 0))
    blk_spec = pl.BlockSpec((B, 1, LC, CB * D), lambda h: (0, h, 0, 0))

    def full(shape):
        return pl.BlockSpec(shape, lambda h: (0,) * len(shape))

    return pl.pallas_call(
        _attn_kernel,
        grid=(H,),
        in_specs=[head_spec, head_spec, head_spec, blk_spec, blk_spec,
                  full((D // 2, CB * D)), full((1, D // 2)),
                  full((D, D // 2)), full((1, D)),
                  full((3, D)), full((3, 1))],
        out_specs=[head_spec],
        out_shape=[jax.ShapeDtypeStruct((B, H, D, L), jnp.float32)],
    )(qT.reshape(B, H, D, L), kT.reshape(B, H, D, L), vT.reshape(B, H, D, L),
      kbT, vbT, Wc1, bc1.reshape(1, D // 2), Wc2, bc2.reshape(1, D),
      Wg, bg.reshape(3, 1))[0]


def _mlp_kernel(x_ref, a_ref, nw_ref, wg_ref, wu_ref, wd_ref, o_ref):
    x2 = x_ref[...] + a_ref[...]
    ms = jnp.mean(x2 * x2, axis=1, keepdims=True)
    h = _bf(x2 * jax.lax.rsqrt(ms + EPS) * nw_ref[...])
    g = jnp.dot(h, wg_ref[...], preferred_element_type=jnp.float32)
    u = jnp.dot(h, wu_ref[...], preferred_element_type=jnp.float32)
    act = _bf(g * jax.nn.sigmoid(g) * u)
    o_ref[...] = jnp.dot(act, wd_ref[...], preferred_element_type=jnp.float32) + x2


def _mlp(xf, af, nw, wgT, wuT, wdT):
    grid = (ROWS // RT,)
    row_spec = pl.BlockSpec((RT, E), lambda i: (i, 0))
    return pl.pallas_call(
        _mlp_kernel,
        grid=grid,
        in_specs=[row_spec, row_spec,
                  pl.BlockSpec((1, E), lambda i: (0, 0)),
                  pl.BlockSpec((E, INTER), lambda i: (0, 0)),
                  pl.BlockSpec((E, INTER), lambda i: (0, 0)),
                  pl.BlockSpec((INTER, E), lambda i: (0, 0))],
        out_specs=[row_spec],
        out_shape=[jax.ShapeDtypeStruct((ROWS, E), jnp.float32)],
    )(xf, af, nw.reshape(1, E), wgT, wuT, wdT)[0]


def kernel(x, attn_norm_w, Wq, bq, Wk, bk, Wv, bv, Wc1, bc1, Wc2, bc2, Wg, bg,
           mlp_norm_w, W_gate, W_up, W_down):
    qT = _proj(x, attn_norm_w, Wq, bq, jnp.float32)
    kT = _proj(x, attn_norm_w, Wk, bk, jnp.float32)
    vT = _proj(x, attn_norm_w, _bf(Wv), bv, jnp.bfloat16)

    kbT = (kT.reshape(B, H, D, L).transpose(0, 1, 3, 2)
             .reshape(B, H, LC, CB * D))
    vbT = (vT.reshape(B, H, D, L).transpose(0, 1, 3, 2)
             .reshape(B, H, LC, CB * D))

    attnT = _attention(qT, kT, vT, kbT, vbT, Wc1, bc1, Wc2, bc2, Wg, bg)
    af = attnT.reshape(B, E, L).transpose(0, 2, 1).reshape(ROWS, E)

    xf = x.reshape(ROWS, E)
    out = _mlp(xf, af, mlp_norm_w, _bf(W_gate.T), _bf(W_up.T), _bf(W_down.T))
    return out.reshape(B, L, E)
